# baseline (speedup 1.0000x reference)
import functools

import jax
import jax.numpy as jnp
from jax import lax
from jax.experimental import pallas as pl
from jax.experimental.pallas import tpu as pltpu
from jax.experimental.pallas import tpu_sc as plsc

VOCAB = 100000
HIDDEN = 64
BATCH = 1024
CTX = 20

NC = 2
NS = 16
NW = NC * NS
B_PER_W = BATCH // NW
IDX_PER_W = B_PER_W * CTX
IDX_CHUNK = 128
N_CHUNKS = IDX_PER_W // IDX_CHUNK

VB = 2048


def _sc_gather_sum(idx3, emb):
    mesh = plsc.VectorSubcoreMesh(core_axis_name="c", subcore_axis_name="s")

    @functools.partial(
        pl.kernel,
        mesh=mesh,
        out_type=jax.ShapeDtypeStruct((BATCH, HIDDEN), jnp.float32),
        scratch_types=[
            pltpu.VMEM((N_CHUNKS, IDX_CHUNK), jnp.int32),
            pltpu.VMEM((IDX_PER_W, HIDDEN), jnp.float32),
            pltpu.VMEM((B_PER_W, HIDDEN), jnp.float32),
            pltpu.SemaphoreType.DMA,
        ],
        compiler_params=pltpu.CompilerParams(use_tc_tiling_on_sc=False),
    )
    def sc_fn(idx_hbm, emb_hbm, x_hbm, idx_v, rows_v, x_v, sem):
        wid = lax.axis_index("s") * NC + lax.axis_index("c")
        pltpu.sync_copy(idx_hbm.at[wid], idx_v)
        copies = []
        for ci in range(N_CHUNKS):
            copies.append(
                pltpu.async_copy(
                    emb_hbm.at[idx_v.at[ci]],
                    rows_v.at[pl.ds(ci * IDX_CHUNK, IDX_CHUNK)],
                    sem,
                )
            )
        for cp in copies:
            cp.wait()

        def body(bi, carry):
            base = bi * CTX
            for h in range(HIDDEN // 16):
                sl = pl.ds(h * 16, 16)
                acc = rows_v[base, sl]
                for j in range(1, CTX):
                    acc = acc + rows_v[base + j, sl]
                x_v[bi, sl] = acc
            return carry

        lax.fori_loop(0, B_PER_W, body, 0)
        pltpu.sync_copy(x_v, x_hbm.at[pl.ds(wid * B_PER_W, B_PER_W)])

    return sc_fn(idx3, emb)


def _mm_body(x_ref, w_ref, b_ref, out_ref):
    out_ref[...] = (
        lax.dot_general(
            x_ref[...],
            w_ref[...],
            (((1,), (1,)), ((), ())),
            preferred_element_type=jnp.float32,
        )
        + b_ref[...]
    )


def _tc_matmul(x, W, b2):
    grid = (VOCAB + VB - 1) // VB
    return pl.pallas_call(
        _mm_body,
        grid=(grid,),
        in_specs=[
            pl.BlockSpec((BATCH, HIDDEN), lambda i: (0, 0)),
            pl.BlockSpec((VB, HIDDEN), lambda i: (i, 0)),
            pl.BlockSpec((1, VB), lambda i: (0, i)),
        ],
        out_specs=pl.BlockSpec((BATCH, VB), lambda i: (0, i)),
        out_shape=jax.ShapeDtypeStruct((BATCH, VOCAB), jnp.float32),
    )(x, W, b2)


def kernel(inputs, emb, W, b):
    idx3 = inputs.astype(jnp.int32).reshape(NW, N_CHUNKS, IDX_CHUNK)
    x = _sc_gather_sum(idx3, emb)
    return _tc_matmul(x, W, b.reshape(1, VOCAB))


# speedup vs baseline: 2.7221x; 2.7221x over previous
import functools

import jax
import jax.numpy as jnp
from jax import lax
from jax.experimental import pallas as pl
from jax.experimental.pallas import tpu as pltpu
from jax.experimental.pallas import tpu_sc as plsc

VOCAB = 100000
HIDDEN = 64
BATCH = 1024
CTX = 20

NC = 2
NS = 16
NW = NC * NS
B_PER_W = BATCH // NW
IDX_PER_W = B_PER_W * CTX
IDX_CHUNK = 128
N_CHUNKS = IDX_PER_W // IDX_CHUNK

VB = 2048


def _sc_gather_sum(idx3, emb):
    mesh = plsc.VectorSubcoreMesh(core_axis_name="c", subcore_axis_name="s")

    @functools.partial(
        pl.kernel,
        mesh=mesh,
        out_type=jax.ShapeDtypeStruct((BATCH, HIDDEN), jnp.float32),
        scratch_types=[
            pltpu.VMEM((N_CHUNKS, IDX_CHUNK), jnp.int32),
            pltpu.VMEM((IDX_PER_W, HIDDEN), jnp.float32),
            pltpu.VMEM((B_PER_W, HIDDEN), jnp.float32),
            pltpu.SemaphoreType.DMA,
        ],
        compiler_params=pltpu.CompilerParams(use_tc_tiling_on_sc=False),
    )
    def sc_fn(idx_hbm, emb_hbm, x_hbm, idx_v, rows_v, x_v, sem):
        wid = lax.axis_index("s") * NC + lax.axis_index("c")
        pltpu.sync_copy(idx_hbm.at[wid], idx_v)
        copies = []
        for ci in range(N_CHUNKS):
            copies.append(
                pltpu.async_copy(
                    emb_hbm.at[idx_v.at[ci]],
                    rows_v.at[pl.ds(ci * IDX_CHUNK, IDX_CHUNK)],
                    sem,
                )
            )
        for cp in copies:
            cp.wait()

        def body(bi, carry):
            base = bi * CTX
            for h in range(HIDDEN // 16):
                sl = pl.ds(h * 16, 16)
                acc = rows_v[base, sl]
                for j in range(1, CTX):
                    acc = acc + rows_v[base + j, sl]
                x_v[bi, sl] = acc
            return carry

        lax.fori_loop(0, B_PER_W, body, 0)
        pltpu.sync_copy(x_v, x_hbm.at[pl.ds(wid * B_PER_W, B_PER_W)])

    return sc_fn(idx3, emb)


def _mm_body(wt_ref, x_ref, b_ref, out_ref):
    acc = lax.dot_general(
        wt_ref[...],
        x_ref[...],
        (((0,), (1,)), ((), ())),
        preferred_element_type=jnp.float32,
    )
    out_ref[...] = acc + jnp.transpose(b_ref[...])


def _tc_matmul(x, Wt, b2):
    grid = (VOCAB + VB - 1) // VB
    out_t = pl.pallas_call(
        _mm_body,
        grid=(grid,),
        in_specs=[
            pl.BlockSpec((HIDDEN, VB), lambda i: (0, i)),
            pl.BlockSpec((BATCH, HIDDEN), lambda i: (0, 0)),
            pl.BlockSpec((1, VB), lambda i: (0, i)),
        ],
        out_specs=pl.BlockSpec((VB, BATCH), lambda i: (i, 0)),
        out_shape=jax.ShapeDtypeStruct((VOCAB, BATCH), jnp.float32),
    )(Wt, x, b2)
    return out_t.T


def kernel(inputs, emb, W, b):
    idx3 = inputs.astype(jnp.int32).reshape(NW, N_CHUNKS, IDX_CHUNK)
    x = _sc_gather_sum(idx3, emb)
    return _tc_matmul(x, W.T, b.reshape(1, VOCAB))


# speedup vs baseline: 2.8278x; 1.0388x over previous
import functools

import jax
import jax.numpy as jnp
from jax import lax
from jax.experimental import pallas as pl
from jax.experimental.pallas import tpu as pltpu
from jax.experimental.pallas import tpu_sc as plsc

VOCAB = 100000
HIDDEN = 64
BATCH = 1024
CTX = 20

NC = 2
NS = 16
NW = NC * NS
B_PER_W = BATCH // NW
IDX_PER_W = B_PER_W * CTX
IDX_CHUNK = 128
N_CHUNKS = IDX_PER_W // IDX_CHUNK

VB = 2048


def _sc_gather_sum(idx3, emb128):
    mesh = plsc.VectorSubcoreMesh(core_axis_name="c", subcore_axis_name="s")

    @functools.partial(
        pl.kernel,
        mesh=mesh,
        out_type=jax.ShapeDtypeStruct((BATCH, HIDDEN), jnp.float32),
        scratch_types=[
            pltpu.VMEM((N_CHUNKS, IDX_CHUNK), jnp.int32),
            pltpu.VMEM((IDX_PER_W, 128), jnp.float32),
            pltpu.VMEM((B_PER_W, HIDDEN), jnp.float32),
            pltpu.SemaphoreType.DMA,
        ],
    )
    def sc_fn(idx_hbm, emb_hbm, x_hbm, idx_v, rows_v, x_v, sem):
        wid = lax.axis_index("s") * NC + lax.axis_index("c")
        pltpu.sync_copy(idx_hbm.at[wid], idx_v)
        copies = []
        for ci in range(N_CHUNKS):
            copies.append(
                pltpu.async_copy(
                    emb_hbm.at[idx_v.at[ci]],
                    rows_v.at[pl.ds(ci * IDX_CHUNK, IDX_CHUNK)],
                    sem,
                )
            )
        for cp in copies:
            cp.wait()

        def body(bi, carry):
            base = bi * CTX
            for h in range(HIDDEN // 16):
                sl = pl.ds(h * 16, 16)
                acc = rows_v[base, sl]
                for j in range(1, CTX):
                    acc = acc + rows_v[base + j, sl]
                x_v[bi, sl] = acc
            return carry

        lax.fori_loop(0, B_PER_W, body, 0)
        pltpu.sync_copy(x_v, x_hbm.at[pl.ds(wid * B_PER_W, B_PER_W)])

    return sc_fn(idx3, emb128)


def _mm_body(wt_ref, x_ref, b_ref, out_ref):
    acc = lax.dot_general(
        wt_ref[...],
        x_ref[...],
        (((0,), (1,)), ((), ())),
        preferred_element_type=jnp.float32,
    )
    out_ref[...] = acc + jnp.transpose(b_ref[...])


def _tc_matmul(x, Wt, b2):
    grid = (VOCAB + VB - 1) // VB
    out_t = pl.pallas_call(
        _mm_body,
        grid=(grid,),
        in_specs=[
            pl.BlockSpec((HIDDEN, VB), lambda i: (0, i)),
            pl.BlockSpec((BATCH, HIDDEN), lambda i: (0, 0)),
            pl.BlockSpec((1, VB), lambda i: (0, i)),
        ],
        out_specs=pl.BlockSpec((VB, BATCH), lambda i: (i, 0)),
        out_shape=jax.ShapeDtypeStruct((VOCAB, BATCH), jnp.float32),
    )(Wt, x, b2)
    return out_t.T


def kernel(inputs, emb, W, b):
    idx3 = inputs.astype(jnp.int32).reshape(NW, N_CHUNKS, IDX_CHUNK)
    emb128 = jnp.pad(emb, ((0, 0), (0, 128 - HIDDEN)))
    x = _sc_gather_sum(idx3, emb128)
    return _tc_matmul(x, W.T, b.reshape(1, VOCAB))


# speedup vs baseline: 3.5275x; 1.2474x over previous
import functools

import jax
import jax.numpy as jnp
from jax import lax
from jax.experimental import pallas as pl
from jax.experimental.pallas import tpu as pltpu
from jax.experimental.pallas import tpu_sc as plsc

VOCAB = 100000
HIDDEN = 64
BATCH = 1024
CTX = 20

NC = 2
NS = 16
NW = NC * NS
H_PHASES = HIDDEN // NW

VB = 2048


def _sc_gather_sum_t(idxT, embT):
    mesh = plsc.VectorSubcoreMesh(core_axis_name="c", subcore_axis_name="s")

    @functools.partial(
        pl.kernel,
        mesh=mesh,
        out_type=jax.ShapeDtypeStruct((HIDDEN, BATCH), jnp.float32),
        scratch_types=[
            pltpu.VMEM((CTX, BATCH), jnp.int32),
            pltpu.VMEM((VOCAB,), jnp.float32),
            pltpu.VMEM((BATCH,), jnp.float32),
        ],
        compiler_params=pltpu.CompilerParams(needs_layout_passes=False),
    )
    def sc_fn(idx_hbm, emb_hbm, xt_hbm, idx_v, row_v, xt_v):
        wid = lax.axis_index("s") * NC + lax.axis_index("c")
        pltpu.sync_copy(idx_hbm, idx_v)
        for p in range(H_PHASES):
            h = wid + p * NW
            pltpu.sync_copy(emb_hbm.at[h], row_v)

            def body(c, carry):
                acc = jnp.zeros((16,), jnp.float32)
                for j in range(CTX):
                    iv = idx_v[j, pl.ds(c * 16, 16)]
                    acc = acc + plsc.load_gather(row_v, [iv])
                xt_v[pl.ds(c * 16, 16)] = acc
                return carry

            lax.fori_loop(0, BATCH // 16, body, 0)
            pltpu.sync_copy(xt_v, xt_hbm.at[h])

    return sc_fn(idxT, embT)


def _mm_body(wt_ref, xt_ref, b_ref, out_ref):
    acc = lax.dot_general(
        wt_ref[...],
        xt_ref[...],
        (((0,), (0,)), ((), ())),
        preferred_element_type=jnp.float32,
    )
    out_ref[...] = acc + jnp.transpose(b_ref[...])


def _tc_matmul(xT, Wt, b2):
    grid = (VOCAB + VB - 1) // VB
    out_t = pl.pallas_call(
        _mm_body,
        grid=(grid,),
        in_specs=[
            pl.BlockSpec((HIDDEN, VB), lambda i: (0, i)),
            pl.BlockSpec((HIDDEN, BATCH), lambda i: (0, 0)),
            pl.BlockSpec((1, VB), lambda i: (0, i)),
        ],
        out_specs=pl.BlockSpec((VB, BATCH), lambda i: (i, 0)),
        out_shape=jax.ShapeDtypeStruct((VOCAB, BATCH), jnp.float32),
    )(Wt, xT, b2)
    return out_t.T


def kernel(inputs, emb, W, b):
    idxT = inputs.astype(jnp.int32).T
    xT = _sc_gather_sum_t(idxT, emb.T)
    return _tc_matmul(xT, W.T, b.reshape(1, VOCAB))


# speedup vs baseline: 3.5730x; 1.0129x over previous
import functools

import jax
import jax.numpy as jnp
from jax import lax
from jax.experimental import pallas as pl
from jax.experimental.pallas import tpu as pltpu
from jax.experimental.pallas import tpu_sc as plsc

VOCAB = 100000
HIDDEN = 64
BATCH = 1024
CTX = 20

NC = 2
NS = 16
NW = NC * NS
H_PHASES = HIDDEN // NW

VB = 4096


def _sc_gather_sum_t(idxT, embT):
    mesh = plsc.VectorSubcoreMesh(core_axis_name="c", subcore_axis_name="s")

    @functools.partial(
        pl.kernel,
        mesh=mesh,
        out_type=jax.ShapeDtypeStruct((HIDDEN, BATCH), jnp.float32),
        scratch_types=[
            pltpu.VMEM((CTX, BATCH), jnp.int32),
            pltpu.VMEM((VOCAB,), jnp.float32),
            pltpu.VMEM((BATCH,), jnp.float32),
        ],
        compiler_params=pltpu.CompilerParams(needs_layout_passes=False),
    )
    def sc_fn(idx_hbm, emb_hbm, xt_hbm, idx_v, row_v, xt_v):
        wid = lax.axis_index("s") * NC + lax.axis_index("c")
        pltpu.sync_copy(idx_hbm, idx_v)
        for p in range(H_PHASES):
            h = wid + p * NW
            pltpu.sync_copy(emb_hbm.at[h], row_v)

            def body(c, carry):
                acc = jnp.zeros((16,), jnp.float32)
                for j in range(CTX):
                    iv = idx_v[j, pl.ds(c * 16, 16)]
                    acc = acc + plsc.load_gather(row_v, [iv])
                xt_v[pl.ds(c * 16, 16)] = acc
                return carry

            lax.fori_loop(0, BATCH // 16, body, 0)
            pltpu.sync_copy(xt_v, xt_hbm.at[h])

    return sc_fn(idxT, embT)


def _mm_body(wt_ref, xt_ref, b_ref, out_ref):
    acc = lax.dot_general(
        wt_ref[...],
        xt_ref[...],
        (((0,), (0,)), ((), ())),
        preferred_element_type=jnp.float32,
    )
    out_ref[...] = acc + jnp.transpose(b_ref[...])


def _tc_matmul(xT, Wt, b2):
    grid = (VOCAB + VB - 1) // VB
    out_t = pl.pallas_call(
        _mm_body,
        grid=(grid,),
        in_specs=[
            pl.BlockSpec((HIDDEN, VB), lambda i: (0, i)),
            pl.BlockSpec((HIDDEN, BATCH), lambda i: (0, 0)),
            pl.BlockSpec((1, VB), lambda i: (0, i)),
        ],
        out_specs=pl.BlockSpec((VB, BATCH), lambda i: (i, 0)),
        out_shape=jax.ShapeDtypeStruct((VOCAB, BATCH), jnp.float32),
    )(Wt, xT, b2)
    return out_t.T


def kernel(inputs, emb, W, b):
    idxT = inputs.astype(jnp.int32).T
    xT = _sc_gather_sum_t(idxT, emb.T)
    return _tc_matmul(xT, W.T, b.reshape(1, VOCAB))


# speedup vs baseline: 3.5767x; 1.0010x over previous
import functools

import jax
import jax.numpy as jnp
from jax import lax
from jax.experimental import pallas as pl
from jax.experimental.pallas import tpu as pltpu
from jax.experimental.pallas import tpu_sc as plsc

VOCAB = 100000
HIDDEN = 64
BATCH = 1024
CTX = 20

NC = 2
NS = 16
NW = NC * NS
H_PHASES = HIDDEN // NW

VB = 4608


def _sc_gather_sum_t(idxT, embT):
    mesh = plsc.VectorSubcoreMesh(core_axis_name="c", subcore_axis_name="s")

    @functools.partial(
        pl.kernel,
        mesh=mesh,
        out_type=jax.ShapeDtypeStruct((HIDDEN, BATCH), jnp.float32),
        scratch_types=[
            pltpu.VMEM((CTX, BATCH), jnp.int32),
            pltpu.VMEM((VOCAB,), jnp.float32),
            pltpu.VMEM((BATCH,), jnp.float32),
        ],
        compiler_params=pltpu.CompilerParams(needs_layout_passes=False),
    )
    def sc_fn(idx_hbm, emb_hbm, xt_hbm, idx_v, row_v, xt_v):
        wid = lax.axis_index("s") * NC + lax.axis_index("c")
        pltpu.sync_copy(idx_hbm, idx_v)
        for p in range(H_PHASES):
            h = wid + p * NW
            pltpu.sync_copy(emb_hbm.at[h], row_v)

            def body(c, carry):
                acc = jnp.zeros((16,), jnp.float32)
                for j in range(CTX):
                    iv = idx_v[j, pl.ds(c * 16, 16)]
                    acc = acc + plsc.load_gather(row_v, [iv])
                xt_v[pl.ds(c * 16, 16)] = acc
                return carry

            lax.fori_loop(0, BATCH // 16, body, 0)
            pltpu.sync_copy(xt_v, xt_hbm.at[h])

    return sc_fn(idxT, embT)


def _mm_body(wt_ref, xt_ref, b_ref, out_ref):
    acc = lax.dot_general(
        wt_ref[...],
        xt_ref[...],
        (((0,), (0,)), ((), ())),
        preferred_element_type=jnp.float32,
    )
    out_ref[...] = acc + jnp.transpose(b_ref[...])


def _tc_matmul(xT, Wt, b2):
    grid = (VOCAB + VB - 1) // VB
    out_t = pl.pallas_call(
        _mm_body,
        grid=(grid,),
        in_specs=[
            pl.BlockSpec((HIDDEN, VB), lambda i: (0, i)),
            pl.BlockSpec((HIDDEN, BATCH), lambda i: (0, 0)),
            pl.BlockSpec((1, VB), lambda i: (0, i)),
        ],
        out_specs=pl.BlockSpec((VB, BATCH), lambda i: (i, 0)),
        out_shape=jax.ShapeDtypeStruct((VOCAB, BATCH), jnp.float32),
    )(Wt, xT, b2)
    return out_t.T


def kernel(inputs, emb, W, b):
    idxT = inputs.astype(jnp.int32).T
    xT = _sc_gather_sum_t(idxT, emb.T)
    return _tc_matmul(xT, W.T, b.reshape(1, VOCAB))


# speedup vs baseline: 3.5883x; 1.0032x over previous
import functools

import jax
import jax.numpy as jnp
from jax import lax
from jax.experimental import pallas as pl
from jax.experimental.pallas import tpu as pltpu
from jax.experimental.pallas import tpu_sc as plsc

VOCAB = 100000
HIDDEN = 64
BATCH = 1024
CTX = 20

NC = 2
NS = 16
NW = NC * NS
H_PHASES = HIDDEN // NW

VB = 4096


def _sc_gather_sum_t(idxT, embT):
    mesh = plsc.VectorSubcoreMesh(core_axis_name="c", subcore_axis_name="s")

    @functools.partial(
        pl.kernel,
        mesh=mesh,
        out_type=jax.ShapeDtypeStruct((HIDDEN, BATCH), jnp.float32),
        scratch_types=[
            pltpu.VMEM((CTX, BATCH), jnp.int32),
            pltpu.VMEM((VOCAB,), jnp.float32),
            pltpu.VMEM((H_PHASES, BATCH), jnp.float32),
            pltpu.SemaphoreType.DMA,
            pltpu.SemaphoreType.DMA,
        ],
        compiler_params=pltpu.CompilerParams(needs_layout_passes=False),
    )
    def sc_fn(idx_hbm, emb_hbm, xt_hbm, idx_v, row_v, xt_v, sem_i, sem_r):
        wid = lax.axis_index("s") * NC + lax.axis_index("c")
        cp_idx = pltpu.async_copy(idx_hbm, idx_v, sem_i)
        cp_row = pltpu.async_copy(emb_hbm.at[wid], row_v, sem_r)
        cp_idx.wait()
        outs = []
        for p in range(H_PHASES):
            h = wid + p * NW
            cp_row.wait()

            def body(c, carry):
                acc = jnp.zeros((16,), jnp.float32)
                for j in range(CTX):
                    iv = idx_v[j, pl.ds(c * 16, 16)]
                    acc = acc + plsc.load_gather(row_v, [iv])
                xt_v[p, pl.ds(c * 16, 16)] = acc
                return carry

            lax.fori_loop(0, BATCH // 16, body, 0)
            if p + 1 < H_PHASES:
                cp_row = pltpu.async_copy(emb_hbm.at[h + NW], row_v, sem_r)
            outs.append(pltpu.async_copy(xt_v.at[p], xt_hbm.at[h], sem_i))
        for cp in outs:
            cp.wait()

    return sc_fn(idxT, embT)


def _mm_body(wt_ref, xt_ref, b_ref, out_ref):
    acc = lax.dot_general(
        wt_ref[...],
        xt_ref[...],
        (((0,), (0,)), ((), ())),
        preferred_element_type=jnp.float32,
    )
    out_ref[...] = acc + jnp.transpose(b_ref[...])


def _tc_matmul(xT, Wt, b2):
    grid = (VOCAB + VB - 1) // VB
    out_t = pl.pallas_call(
        _mm_body,
        grid=(grid,),
        in_specs=[
            pl.BlockSpec((HIDDEN, VB), lambda i: (0, i)),
            pl.BlockSpec((HIDDEN, BATCH), lambda i: (0, 0)),
            pl.BlockSpec((1, VB), lambda i: (0, i)),
        ],
        out_specs=pl.BlockSpec((VB, BATCH), lambda i: (i, 0)),
        out_shape=jax.ShapeDtypeStruct((VOCAB, BATCH), jnp.float32),
    )(Wt, xT, b2)
    return out_t.T


def kernel(inputs, emb, W, b):
    idxT = inputs.astype(jnp.int32).T
    xT = _sc_gather_sum_t(idxT, emb.T)
    return _tc_matmul(xT, W.T, b.reshape(1, VOCAB))
